# Initial kernel scaffold; baseline (speedup 1.0000x reference)
import jax
import jax.numpy as jnp
from jax.experimental import pallas as pl


def kernel(rnatok, emb_table):
    raise NotImplementedError("write your pallas kernel here")


# speedup vs baseline: 4.4959x; 4.4959x over previous
import functools

import jax
import jax.numpy as jnp
import numpy as np
from jax import lax
from jax.experimental import pallas as pl
from jax.experimental.pallas import tpu as pltpu
from jax.experimental.pallas import tpu_sc as plsc

D_MODEL = 64
MAXLEN = 200
VOCAB = 5
BATCH = 4096
ROWS = BATCH * MAXLEN
NC, NS = 2, 16
NW = NC * NS
RPW = ROWS // NW
CHUNK = 1024
NCHUNK = RPW // CHUNK
JV = CHUNK // 16
NGATHER = CHUNK // 128
L_PER = 13


def _positional_encoding() -> np.ndarray:
    pos = np.arange(MAXLEN)[:, None]
    i = np.arange(D_MODEL)[None, :]
    rates = 1 / np.power(10000, 2 * (i // 2) / np.float32(D_MODEL))
    angle = pos * rates
    angle[:, 0::2] = np.sin(angle[:, 0::2])
    angle[:, 1::2] = np.cos(angle[:, 1::2])
    return angle.astype(np.float32)


_PE = _positional_encoding()


def _body(tok_hbm, table_hbm, pe_hbm, out_hbm,
          pe_v, tab_v, build_v, fused_sh, tok_v, idx_v, rows_v, sem):
    s = lax.axis_index("s")
    c = lax.axis_index("c")
    wid = s * NC + c

    pltpu.sync_copy(table_hbm, tab_v)
    pltpu.sync_copy(pe_hbm, pe_v)
    for v in range(VOCAB):
        for k in range(D_MODEL // 16):
            tab_v[v, pl.ds(k * 16, 16)] = tab_v[v, pl.ds(k * 16, 16)] * 8.0
    l0 = s * L_PER
    for li in range(L_PER):
        l = l0 + li

        @pl.when(l < MAXLEN)
        def _build():
            for v in range(VOCAB):
                for k in range(D_MODEL // 16):
                    sl = pl.ds(k * 16, 16)
                    build_v[v, sl] = (tab_v[v, sl] + pe_v[l, sl]) * 0.125
            pltpu.sync_copy(build_v, fused_sh.at[pl.ds(l * VOCAB, VOCAB)])

    plsc.subcore_barrier()

    iota16 = lax.broadcasted_iota(jnp.int32, (16,), 0)
    base_w = wid * RPW

    @pl.loop(0, NCHUNK)
    def _chunk(g):
        base = base_w + g * CHUNK
        pltpu.sync_copy(tok_hbm.at[pl.ds(base, CHUNK)], tok_v)
        off = g * CHUNK
        for j in range(JV):
            pos = lax.rem(off + j * 16 + iota16, MAXLEN)
            idx = pos * VOCAB + tok_v[pl.ds(j * 16, 16)]
            idx_v[j // 8, pl.ds((j % 8) * 16, 16)] = idx
        copies = []
        for r in range(NGATHER):
            copies.append(pltpu.async_copy(
                fused_sh.at[idx_v.at[r]],
                rows_v.at[pl.ds(r * 128, 128)], sem))
        for cp in copies:
            cp.wait()
        pltpu.sync_copy(rows_v, out_hbm.at[pl.ds(base, CHUNK)])


@functools.partial(jax.jit, static_argnames=())
def _sc_embed(tok_flat, emb_table, pe):
    mesh = plsc.VectorSubcoreMesh(core_axis_name="c", subcore_axis_name="s",
                                  num_cores=NC, num_subcores=NS)
    return pl.kernel(
        _body,
        out_type=jax.ShapeDtypeStruct((ROWS, D_MODEL), jnp.float32),
        mesh=mesh,
        scratch_types=[
            pltpu.VMEM((MAXLEN, D_MODEL), jnp.float32),
            pltpu.VMEM((VOCAB, D_MODEL), jnp.float32),
            pltpu.VMEM((VOCAB, D_MODEL), jnp.float32),
            pltpu.VMEM_SHARED((MAXLEN * VOCAB, D_MODEL), jnp.float32),
            pltpu.VMEM((CHUNK,), jnp.int32),
            pltpu.VMEM((NGATHER, 128), jnp.int32),
            pltpu.VMEM((CHUNK, D_MODEL), jnp.float32),
            pltpu.SemaphoreType.DMA,
        ],
        compiler_params=pltpu.CompilerParams(use_tc_tiling_on_sc=False),
    )(tok_flat, emb_table, pe)


def kernel(rnatok, emb_table):
    tok_flat = rnatok.reshape(ROWS).astype(jnp.int32)
    pe = jnp.asarray(_PE)
    out = _sc_embed(tok_flat, emb_table, pe)
    return out.reshape(BATCH, MAXLEN, D_MODEL)
